# initial kernel scaffold (unmeasured)
import jax
import jax.numpy as jnp
from jax import lax
from jax.experimental import pallas as pl
from jax.experimental.pallas import tpu as pltpu


def kernel(
    x,
):
    def body(*refs):
        pass

    out_shape = jax.ShapeDtypeStruct(..., jnp.float32)
    return pl.pallas_call(body, out_shape=out_shape)(...)


# baseline (device time: 37430 ns/iter reference)
import jax
import jax.numpy as jnp
from jax import lax
from jax.experimental import pallas as pl
from jax.experimental.pallas import tpu as pltpu

N_DEV = 16
K = 8
NEG = -1e30


def kernel(x):
    m, n = x.shape

    def body(x_ref, out_ref, cand_ref, send_sems, recv_sems):
        my = lax.axis_index("i")

        vals = x_ref[:, :]
        cols = []
        for j in range(K):
            mx = jnp.max(vals, axis=1, keepdims=True)
            cols.append(mx)
            if j < K - 1:
                vals = jnp.where(vals == mx, NEG, vals)
        cand_ref[N_DEV - 1, :, :] = jnp.concatenate(cols, axis=1)

        rdmas = []
        for off in range(1, N_DEV):
            dst = (my + off) % N_DEV
            rdma = pltpu.make_async_remote_copy(
                src_ref=cand_ref.at[N_DEV - 1],
                dst_ref=cand_ref.at[off - 1],
                send_sem=send_sems.at[off - 1],
                recv_sem=recv_sems.at[off - 1],
                device_id=(dst,),
                device_id_type=pl.DeviceIdType.MESH,
            )
            rdma.start()
            rdmas.append(rdma)
        for rdma in rdmas:
            rdma.wait()

        allv = cand_ref[:, :, :]
        outs = []
        for j in range(K):
            mx = jnp.max(jnp.max(allv, axis=2), axis=0)
            outs.append(mx[:, None])
            if j < K - 1:
                allv = jnp.where(allv == mx[None, :, None], NEG, allv)
        out_ref[:, :] = jnp.concatenate(outs, axis=1)

    return pl.pallas_call(
        body,
        out_shape=jax.ShapeDtypeStruct((m, K), jnp.float32),
        in_specs=[pl.BlockSpec(memory_space=pltpu.VMEM)],
        out_specs=pl.BlockSpec(memory_space=pltpu.VMEM),
        scratch_shapes=[
            pltpu.VMEM((N_DEV, m, K), jnp.float32),
            pltpu.SemaphoreType.DMA((N_DEV - 1,)),
            pltpu.SemaphoreType.DMA((N_DEV - 1,)),
        ],
    )(x)


# device time: 18345 ns/iter; 2.0403x vs baseline; 2.0403x over previous
import jax
import jax.numpy as jnp
from jax import lax
from jax.experimental import pallas as pl
from jax.experimental.pallas import tpu as pltpu

N_DEV = 16
K = 8
NEG = -1e30


def kernel(x):
    m, n = x.shape

    def body(x_ref, out_ref, cand_ref, send_sems, recv_sems):
        my = lax.axis_index("i")

        vals = x_ref[:, :]
        for j in range(K):
            mx = jnp.max(vals, axis=1)
            cand_ref[N_DEV - 1, j, :] = mx
            if j < K - 1:
                vals = jnp.where(vals == mx[:, None], NEG, vals)

        rdmas = []
        for off in range(1, N_DEV):
            dst = (my + off) % N_DEV
            rdma = pltpu.make_async_remote_copy(
                src_ref=cand_ref.at[N_DEV - 1],
                dst_ref=cand_ref.at[off - 1],
                send_sem=send_sems.at[off - 1],
                recv_sem=recv_sems.at[off - 1],
                device_id=(dst,),
                device_id_type=pl.DeviceIdType.MESH,
            )
            rdma.start()
            rdmas.append(rdma)
        for rdma in rdmas:
            rdma.wait()

        allv = cand_ref[:, :, :].reshape(N_DEV * K, m)
        for j in range(K):
            mx = jnp.max(allv, axis=0)
            out_ref[:, j] = mx
            if j < K - 1:
                allv = jnp.where(allv == mx[None, :], NEG, allv)

    return pl.pallas_call(
        body,
        out_shape=jax.ShapeDtypeStruct((m, K), jnp.float32),
        in_specs=[pl.BlockSpec(memory_space=pltpu.VMEM)],
        out_specs=pl.BlockSpec(memory_space=pltpu.VMEM),
        scratch_shapes=[
            pltpu.VMEM((N_DEV, K, m), jnp.float32),
            pltpu.SemaphoreType.DMA((N_DEV - 1,)),
            pltpu.SemaphoreType.DMA((N_DEV - 1,)),
        ],
    )(x)
